# Initial kernel scaffold; baseline (speedup 1.0000x reference)
import jax
import jax.numpy as jnp
from jax.experimental import pallas as pl


def kernel(x, params):
    raise NotImplementedError("write your pallas kernel here")


# speedup vs baseline: 17.4193x; 17.4193x over previous
import functools

import jax
import jax.numpy as jnp
from jax import lax
from jax.experimental import pallas as pl
from jax.experimental.pallas import tpu as pltpu
from jax.experimental.pallas import tpu_sc as plsc

D = 768
SS = 16
E = 4
DFF = 4 * D
L = 2048

CHUNK = 64
TT = 512
FF = 1536
NF = DFF // FF
NT = L // TT
VT = 1280


def _emb_gather_sc(table, idx):
  V, Dd = table.shape
  B = idx.shape[0]
  info = plsc.get_sparse_core_info()
  NC, NS = info.num_cores, info.num_subcores
  NW = NC * NS
  b_per_w = B // NW
  mesh = plsc.VectorSubcoreMesh(core_axis_name="c", subcore_axis_name="s")

  @functools.partial(
      pl.kernel, mesh=mesh,
      out_type=jax.ShapeDtypeStruct((B, Dd), jnp.float32),
      scratch_types=[
          pltpu.VMEM((b_per_w,), jnp.int32),
          pltpu.VMEM((b_per_w, Dd), jnp.float32),
          pltpu.SemaphoreType.DMA,
      ],
  )
  def k(table_hbm, idx_hbm, out_hbm, idx_v, rows_v, sem):
    wid = lax.axis_index("s") * NC + lax.axis_index("c")
    base = wid * b_per_w
    pltpu.sync_copy(idx_hbm.at[pl.ds(base, b_per_w)], idx_v)
    pltpu.async_copy(table_hbm.at[idx_v], rows_v, sem).wait()
    pltpu.sync_copy(rows_v, out_hbm.at[pl.ds(base, b_per_w)])

  return k(table, idx)


def _ssm_body(x_ref, dw_ref, db_ref, bw_ref, bb_ref, cw_ref, cb_ref,
              alt_ref, dp_ref, rw_ref, rb_ref,
              out_ref, tw_ref, ti_ref,
              delta_s, bm_s, cm_s, ba_s, bb_s, hs_s):
  x = x_ref[...]
  delta_s[...] = jax.nn.softplus(
      jnp.dot(x, dw_ref[...], preferred_element_type=jnp.float32) + db_ref[...])
  bm_s[...] = jnp.dot(x, bw_ref[...], preferred_element_type=jnp.float32) + bb_ref[...]
  cm_s[...] = jnp.dot(x, cw_ref[...], preferred_element_type=jnp.float32) + cb_ref[...]
  a_t = -jnp.exp(alt_ref[...])

  def chunk_body(c, h):
    t0 = c * CHUNK
    dch = delta_s[pl.ds(t0, CHUNK), :]
    bch = bm_s[pl.ds(t0, CHUNK), :]
    ba_s[...] = jnp.exp(jnp.minimum(dch[:, None, :] * a_t[None, :, :], 2.0))
    bb_s[...] = jnp.clip(dch[:, None, :] * bch[:, :, None], -2.0, 2.0)

    def step(t, hc):
      xrow = x_ref[pl.ds(t0 + t, 1), :]
      hc = ba_s[t] * hc + bb_s[t] * xrow
      hc = jnp.clip(hc, -100.0, 100.0)
      hs_s[t] = hc
      return hc

    h = lax.fori_loop(0, CHUNK, step, h)
    cch = cm_s[pl.ds(t0, CHUNK), :]
    xch = x_ref[pl.ds(t0, CHUNK), :]
    y = jnp.sum(hs_s[...] * cch[:, :, None], axis=1)
    out_ref[pl.ds(t0, CHUNK), :] = y + xch * dp_ref[...]
    return h

  h0 = jnp.zeros((SS, D), jnp.float32)
  lax.fori_loop(0, L // CHUNK, chunk_body, h0)

  so = out_ref[...]
  logits = jnp.dot(so, rw_ref[...], preferred_element_type=jnp.float32) + rb_ref[...]
  m = jnp.max(logits, axis=1, keepdims=True)
  s = jnp.sum(jnp.exp(logits - m), axis=1, keepdims=True)
  tw_ref[...] = 1.0 / s
  lane = lax.broadcasted_iota(jnp.int32, logits.shape, 1)
  ti_ref[...] = jnp.min(jnp.where(logits == m, lane, E), axis=1, keepdims=True)


def _ssm_call(x, p):
  out_shapes = (
      jax.ShapeDtypeStruct((L, D), jnp.float32),
      jax.ShapeDtypeStruct((L, 1), jnp.float32),
      jax.ShapeDtypeStruct((L, 1), jnp.int32),
  )
  scratch = [
      pltpu.VMEM((L, D), jnp.float32),
      pltpu.VMEM((L, SS), jnp.float32),
      pltpu.VMEM((L, SS), jnp.float32),
      pltpu.VMEM((CHUNK, SS, D), jnp.float32),
      pltpu.VMEM((CHUNK, SS, D), jnp.float32),
      pltpu.VMEM((CHUNK, SS, D), jnp.float32),
  ]
  return pl.pallas_call(
      _ssm_body,
      out_shape=out_shapes,
      scratch_shapes=scratch,
  )(x, p['delta_w'], p['delta_b'].reshape(1, D),
    p['B_w'], p['B_b'].reshape(1, SS),
    p['C_w'], p['C_b'].reshape(1, SS),
    p['A_log'].T, p['Dp'].reshape(1, D),
    p['router_w'], p['router_b'].reshape(1, E))


def _split3(x):
  h1 = x.astype(jnp.bfloat16)
  r1 = x - h1.astype(jnp.float32)
  h2 = r1.astype(jnp.bfloat16)
  h3 = (r1 - h2.astype(jnp.float32)).astype(jnp.bfloat16)
  return h1, h2, h3


def _onehot_gather_exact(p_bf, x):
  h1, h2, h3 = _split3(x)
  d1 = jnp.dot(p_bf, h1, preferred_element_type=jnp.float32)
  d2 = jnp.dot(p_bf, h2, preferred_element_type=jnp.float32)
  d3 = jnp.dot(p_bf, h3, preferred_element_type=jnp.float32)
  return (d1 + d2) + d3


def _onehot_scatter_exact(p_bf, x):
  h1, h2, h3 = _split3(x)
  dn = (((0,), (0,)), ((), ()))
  d1 = lax.dot_general(p_bf, h1, dn, preferred_element_type=jnp.float32)
  d2 = lax.dot_general(p_bf, h2, dn, preferred_element_type=jnp.float32)
  d3 = lax.dot_general(p_bf, h3, dn, preferred_element_type=jnp.float32)
  return (d1 + d2) + d3


def _moe_body(offs_ref, flat_ref, perm_ref, tw_ref,
              uw_ref, ub_ref, dw_ref, db_ref, g_ref, b_ref,
              out_ref, p_s, xg_s, wg_s, yacc_s):
  e = pl.program_id(0)
  j = pl.program_id(1)
  f = pl.program_id(2)
  base = offs_ref[e] + j * TT
  end = offs_ref[e + 1]
  active = base < end
  base_c = jnp.minimum(base, L - TT)

  @pl.when((e == 0) & (j == 0) & (f == 0))
  def _():
    out_ref[...] = jnp.zeros_like(out_ref)

  @pl.when(active & (f == 0))
  def _():
    perm_sl = perm_ref[pl.ds(base_c, TT), :]
    sub = lax.broadcasted_iota(jnp.int32, (TT, 1), 0)
    pos = base_c + sub
    valid = (pos >= base) & (pos < end)
    lane = lax.broadcasted_iota(jnp.int32, (TT, L), 1)
    p_s[...] = jnp.where(valid & (perm_sl == lane), 1.0, 0.0)
    p_bf = p_s[...].astype(jnp.bfloat16)
    xg_s[...] = _onehot_gather_exact(p_bf, flat_ref[...])
    wg_s[...] = _onehot_gather_exact(p_bf, tw_ref[...])

  @pl.when(active)
  def _():
    hid = jnp.dot(xg_s[...], uw_ref[0], preferred_element_type=jnp.float32) + ub_ref[0]
    hid = hid * jax.nn.sigmoid(hid)
    part = jnp.dot(hid, dw_ref[0], preferred_element_type=jnp.float32)

    @pl.when(f == 0)
    def _():
      yacc_s[...] = part

    @pl.when(f != 0)
    def _():
      yacc_s[...] = yacc_s[...] + part

  @pl.when(active & (f == NF - 1))
  def _():
    ysel = (yacc_s[...] + db_ref[0]).astype(jnp.bfloat16).astype(jnp.float32)
    rows = xg_s[...] + ysel * wg_s[...]
    mu = jnp.mean(rows, axis=1, keepdims=True)
    dev = rows - mu
    var = jnp.mean(dev * dev, axis=1, keepdims=True)
    ln = dev / jnp.sqrt(var + 1e-5) * g_ref[...] + b_ref[...]
    out_ref[...] = out_ref[...] + _onehot_scatter_exact(
        p_s[...].astype(jnp.bfloat16), ln)


def _moe_call(flat, perm, offs, tw, p):
  grid = (E, NT, NF)
  return pl.pallas_call(
      _moe_body,
      grid=grid,
      in_specs=[
          pl.BlockSpec(memory_space=pltpu.SMEM),
          pl.BlockSpec((L, D), lambda e, j, f: (0, 0)),
          pl.BlockSpec((L, 1), lambda e, j, f: (0, 0)),
          pl.BlockSpec((L, 1), lambda e, j, f: (0, 0)),
          pl.BlockSpec((1, D, FF), lambda e, j, f: (e, 0, f)),
          pl.BlockSpec((1, 1, FF), lambda e, j, f: (e, 0, f)),
          pl.BlockSpec((1, FF, D), lambda e, j, f: (e, f, 0)),
          pl.BlockSpec((1, 1, D), lambda e, j, f: (e, 0, 0)),
          pl.BlockSpec((1, D), lambda e, j, f: (0, 0)),
          pl.BlockSpec((1, D), lambda e, j, f: (0, 0)),
      ],
      out_specs=pl.BlockSpec((L, D), lambda e, j, f: (0, 0)),
      out_shape=jax.ShapeDtypeStruct((L, D), jnp.float32),
      scratch_shapes=[
          pltpu.VMEM((TT, L), jnp.float32),
          pltpu.VMEM((TT, D), jnp.float32),
          pltpu.VMEM((TT, 1), jnp.float32),
          pltpu.VMEM((TT, D), jnp.float32),
      ],
  )(offs, flat, perm, tw,
    p['up_w'], p['up_b'].reshape(E, 1, DFF), p['down_w'],
    p['down_b'].reshape(E, 1, D),
    p['ln_g'].reshape(1, D), p['ln_b'].reshape(1, D))


def _unembed_body(h_ref, emb_ref, out_ref):
  out_ref[...] = lax.dot_general(
      h_ref[...], emb_ref[...], (((1,), (1,)), ((), ())),
      preferred_element_type=jnp.float32)


def _unembed_call(h, embed):
  V = embed.shape[0]
  return pl.pallas_call(
      _unembed_body,
      grid=(V // VT,),
      in_specs=[
          pl.BlockSpec((L, D), lambda v: (0, 0)),
          pl.BlockSpec((VT, D), lambda v: (v, 0)),
      ],
      out_specs=pl.BlockSpec((L, VT), lambda v: (0, v)),
      out_shape=jax.ShapeDtypeStruct((L, V), jnp.float32),
  )(h, embed)


def _layer(lp, h):
  ssm_out, tw, ti = _ssm_call(h, lp)
  ti_flat = ti.reshape(L)
  perm = jnp.argsort(ti_flat).astype(jnp.int32)
  counts = jnp.bincount(ti_flat, length=E)
  offs = jnp.concatenate([jnp.zeros((1,), jnp.int32),
                          jnp.cumsum(counts).astype(jnp.int32)])
  return _moe_call(ssm_out, perm.reshape(L, 1), offs, tw, lp)


def kernel(x, params):
  idx = x.reshape(L).astype(jnp.int32)
  h = _emb_gather_sc(params['embed'], idx)
  for lp in params['layers']:
    h = _layer(lp, h)
  logits = _unembed_call(h, params['embed'])
  return logits.reshape(1, L, -1)


# speedup vs baseline: 19.1084x; 1.0970x over previous
import functools

import jax
import jax.numpy as jnp
from jax import lax
from jax.experimental import pallas as pl
from jax.experimental.pallas import tpu as pltpu
from jax.experimental.pallas import tpu_sc as plsc

D = 768
SS = 16
E = 4
DFF = 4 * D
L = 2048

CHUNK = 64
TT = 512
FF = 1536
NF = DFF // FF
NT = L // TT
VT = 1280


def _emb_gather_sc(table, idx):
  V, Dd = table.shape
  B = idx.shape[0]
  info = plsc.get_sparse_core_info()
  NC, NS = info.num_cores, info.num_subcores
  NW = NC * NS
  b_per_w = B // NW
  mesh = plsc.VectorSubcoreMesh(core_axis_name="c", subcore_axis_name="s")

  @functools.partial(
      pl.kernel, mesh=mesh,
      out_type=jax.ShapeDtypeStruct((B, Dd), jnp.float32),
      scratch_types=[
          pltpu.VMEM((b_per_w,), jnp.int32),
          pltpu.VMEM((b_per_w, Dd), jnp.float32),
          pltpu.SemaphoreType.DMA,
      ],
  )
  def k(table_hbm, idx_hbm, out_hbm, idx_v, rows_v, sem):
    wid = lax.axis_index("s") * NC + lax.axis_index("c")
    base = wid * b_per_w
    pltpu.sync_copy(idx_hbm.at[pl.ds(base, b_per_w)], idx_v)
    pltpu.async_copy(table_hbm.at[idx_v], rows_v, sem).wait()
    pltpu.sync_copy(rows_v, out_hbm.at[pl.ds(base, b_per_w)])

  return k(table, idx)


def _ssm_body(x_ref, dw_ref, db_ref, bw_ref, bb_ref, cw_ref, cb_ref,
              alt_ref, dp_ref, rw_ref, rb_ref,
              out_ref, tw_ref, ti_ref,
              delta_s, bm_s, cm_s, ba_s, bb_s, hs_s):
  x = x_ref[...]
  delta_s[...] = jax.nn.softplus(
      jnp.dot(x, dw_ref[...], preferred_element_type=jnp.float32) + db_ref[...])
  bm_s[...] = jnp.dot(x, bw_ref[...], preferred_element_type=jnp.float32) + bb_ref[...]
  cm_s[...] = jnp.dot(x, cw_ref[...], preferred_element_type=jnp.float32) + cb_ref[...]
  a_t = -jnp.exp(alt_ref[...])

  def chunk_body(c, h):
    t0 = c * CHUNK
    dch = delta_s[pl.ds(t0, CHUNK), :]
    bch = bm_s[pl.ds(t0, CHUNK), :]
    ba_s[...] = jnp.exp(jnp.minimum(dch[:, None, :] * a_t[None, :, :], 2.0))
    bb_s[...] = jnp.clip(dch[:, None, :] * bch[:, :, None], -2.0, 2.0)

    def step(t, hc):
      xrow = x_ref[pl.ds(t0 + t, 1), :]
      hc = ba_s[t] * hc + bb_s[t] * xrow
      hc = jnp.clip(hc, -100.0, 100.0)
      hs_s[t] = hc
      return hc

    h = lax.fori_loop(0, CHUNK, step, h)
    cch = cm_s[pl.ds(t0, CHUNK), :]
    xch = x_ref[pl.ds(t0, CHUNK), :]
    y = jnp.sum(hs_s[...] * cch[:, :, None], axis=1)
    out_ref[pl.ds(t0, CHUNK), :] = y + xch * dp_ref[...]
    return h

  h0 = jnp.zeros((SS, D), jnp.float32)
  lax.fori_loop(0, L // CHUNK, chunk_body, h0)

  so = out_ref[...]
  logits = jnp.dot(so, rw_ref[...], preferred_element_type=jnp.float32) + rb_ref[...]
  m = jnp.max(logits, axis=1, keepdims=True)
  s = jnp.sum(jnp.exp(logits - m), axis=1, keepdims=True)
  tw_ref[...] = 1.0 / s
  lane = lax.broadcasted_iota(jnp.int32, logits.shape, 1)
  ti_ref[...] = jnp.min(jnp.where(logits == m, lane, E), axis=1, keepdims=True)


def _ssm_call(x, p):
  out_shapes = (
      jax.ShapeDtypeStruct((L, D), jnp.float32),
      jax.ShapeDtypeStruct((L, 1), jnp.float32),
      jax.ShapeDtypeStruct((L, 1), jnp.int32),
  )
  scratch = [
      pltpu.VMEM((L, D), jnp.float32),
      pltpu.VMEM((L, SS), jnp.float32),
      pltpu.VMEM((L, SS), jnp.float32),
      pltpu.VMEM((CHUNK, SS, D), jnp.float32),
      pltpu.VMEM((CHUNK, SS, D), jnp.float32),
      pltpu.VMEM((CHUNK, SS, D), jnp.float32),
  ]
  return pl.pallas_call(
      _ssm_body,
      out_shape=out_shapes,
      scratch_shapes=scratch,
  )(x, p['delta_w'], p['delta_b'].reshape(1, D),
    p['B_w'], p['B_b'].reshape(1, SS),
    p['C_w'], p['C_b'].reshape(1, SS),
    p['A_log'].T, p['Dp'].reshape(1, D),
    p['router_w'], p['router_b'].reshape(1, E))


def _split3(x):
  h1 = x.astype(jnp.bfloat16)
  r1 = x - h1.astype(jnp.float32)
  h2 = r1.astype(jnp.bfloat16)
  h3 = (r1 - h2.astype(jnp.float32)).astype(jnp.bfloat16)
  return h1, h2, h3


def _onehot_gather_exact(p_bf, x):
  h1, h2, h3 = _split3(x)
  d1 = jnp.dot(p_bf, h1, preferred_element_type=jnp.float32)
  d2 = jnp.dot(p_bf, h2, preferred_element_type=jnp.float32)
  d3 = jnp.dot(p_bf, h3, preferred_element_type=jnp.float32)
  return (d1 + d2) + d3


def _onehot_scatter_exact(p_bf, x):
  h1, h2, h3 = _split3(x)
  dn = (((0,), (0,)), ((), ()))
  d1 = lax.dot_general(p_bf, h1, dn, preferred_element_type=jnp.float32)
  d2 = lax.dot_general(p_bf, h2, dn, preferred_element_type=jnp.float32)
  d3 = lax.dot_general(p_bf, h3, dn, preferred_element_type=jnp.float32)
  return (d1 + d2) + d3


def _build_onehot(perm_ref, base, base_c, end):
  perm_sl = perm_ref[pl.ds(base_c, TT), :]
  sub = lax.broadcasted_iota(jnp.int32, (TT, 1), 0)
  pos = base_c + sub
  valid = (pos >= base) & (pos < end)
  lane = lax.broadcasted_iota(jnp.int32, (TT, L), 1)
  return jnp.where(valid & (perm_sl == lane), 1.0, 0.0).astype(jnp.bfloat16)


def _moe_body(offs_ref, flat_ref, perm_ref, tw_ref,
              uw_ref, ub_ref, dw_ref, db_ref, g_ref, b_ref,
              out_ref, xg_s, wg_s, yacc_s):
  e = pl.program_id(0)
  f = pl.program_id(1)
  j = pl.program_id(2)
  base = offs_ref[e] + j * TT
  end = offs_ref[e + 1]
  active = base < end
  base_c = jnp.minimum(base, L - TT)

  @pl.when((e == 0) & (f == 0) & (j == 0))
  def _():
    out_ref[...] = jnp.zeros_like(out_ref)

  @pl.when(active & (f == 0))
  def _():
    p_bf = _build_onehot(perm_ref, base, base_c, end)
    xg_s[j] = _onehot_gather_exact(p_bf, flat_ref[...])
    wg_s[j] = _onehot_gather_exact(p_bf, tw_ref[...])

  @pl.when(active)
  def _():
    hid = jnp.dot(xg_s[j], uw_ref[0], preferred_element_type=jnp.float32) + ub_ref[0]
    hid = hid * jax.nn.sigmoid(hid)
    part = jnp.dot(hid, dw_ref[0], preferred_element_type=jnp.float32)

    @pl.when(f == 0)
    def _():
      yacc_s[j] = part

    @pl.when(f != 0)
    def _():
      yacc_s[j] = yacc_s[j] + part

  @pl.when(active & (f == NF - 1))
  def _():
    ysel = (yacc_s[j] + db_ref[0]).astype(jnp.bfloat16).astype(jnp.float32)
    rows = xg_s[j] + ysel * wg_s[j]
    mu = jnp.mean(rows, axis=1, keepdims=True)
    dev = rows - mu
    var = jnp.mean(dev * dev, axis=1, keepdims=True)
    ln = dev / jnp.sqrt(var + 1e-5) * g_ref[...] + b_ref[...]
    p_bf = _build_onehot(perm_ref, base, base_c, end)
    out_ref[...] = out_ref[...] + _onehot_scatter_exact(p_bf, ln)


def _moe_call(flat, perm, offs, tw, p):
  grid = (E, NF, NT)
  return pl.pallas_call(
      _moe_body,
      grid=grid,
      in_specs=[
          pl.BlockSpec(memory_space=pltpu.SMEM),
          pl.BlockSpec((L, D), lambda e, f, j: (0, 0)),
          pl.BlockSpec((L, 1), lambda e, f, j: (0, 0)),
          pl.BlockSpec((L, 1), lambda e, f, j: (0, 0)),
          pl.BlockSpec((1, D, FF), lambda e, f, j: (e, 0, f)),
          pl.BlockSpec((1, 1, FF), lambda e, f, j: (e, 0, f)),
          pl.BlockSpec((1, FF, D), lambda e, f, j: (e, f, 0)),
          pl.BlockSpec((1, 1, D), lambda e, f, j: (e, 0, 0)),
          pl.BlockSpec((1, D), lambda e, f, j: (0, 0)),
          pl.BlockSpec((1, D), lambda e, f, j: (0, 0)),
      ],
      out_specs=pl.BlockSpec((L, D), lambda e, f, j: (0, 0)),
      out_shape=jax.ShapeDtypeStruct((L, D), jnp.float32),
      scratch_shapes=[
          pltpu.VMEM((NT, TT, D), jnp.float32),
          pltpu.VMEM((NT, TT, 1), jnp.float32),
          pltpu.VMEM((NT, TT, D), jnp.float32),
      ],
  )(offs, flat, perm, tw,
    p['up_w'], p['up_b'].reshape(E, 1, DFF), p['down_w'],
    p['down_b'].reshape(E, 1, D),
    p['ln_g'].reshape(1, D), p['ln_b'].reshape(1, D))


def _unembed_body(h_ref, emb_ref, out_ref):
  out_ref[...] = lax.dot_general(
      h_ref[...], emb_ref[...], (((1,), (1,)), ((), ())),
      preferred_element_type=jnp.float32)


def _unembed_call(h, embed):
  V = embed.shape[0]
  return pl.pallas_call(
      _unembed_body,
      grid=(V // VT,),
      in_specs=[
          pl.BlockSpec((L, D), lambda v: (0, 0)),
          pl.BlockSpec((VT, D), lambda v: (v, 0)),
      ],
      out_specs=pl.BlockSpec((L, VT), lambda v: (0, v)),
      out_shape=jax.ShapeDtypeStruct((L, V), jnp.float32),
  )(h, embed)


def _layer(lp, h):
  ssm_out, tw, ti = _ssm_call(h, lp)
  ti_flat = ti.reshape(L)
  perm = jnp.argsort(ti_flat).astype(jnp.int32)
  counts = jnp.bincount(ti_flat, length=E)
  offs = jnp.concatenate([jnp.zeros((1,), jnp.int32),
                          jnp.cumsum(counts).astype(jnp.int32)])
  return _moe_call(ssm_out, perm.reshape(L, 1), offs, tw, lp)


def kernel(x, params):
  idx = x.reshape(L).astype(jnp.int32)
  h = _emb_gather_sc(params['embed'], idx)
  for lp in params['layers']:
    h = _layer(lp, h)
  logits = _unembed_call(h, params['embed'])
  return logits.reshape(1, L, -1)


# speedup vs baseline: 19.2861x; 1.0093x over previous
import functools

import jax
import jax.numpy as jnp
from jax import lax
from jax.experimental import pallas as pl
from jax.experimental.pallas import tpu as pltpu
from jax.experimental.pallas import tpu_sc as plsc

D = 768
SS = 16
E = 4
DFF = 4 * D
L = 2048

CHUNK = 128
TT = 512
FF = 1536
NF = DFF // FF
NT = L // TT
VT = 1280


def _emb_gather_sc(table, idx):
  V, Dd = table.shape
  B = idx.shape[0]
  info = plsc.get_sparse_core_info()
  NC, NS = info.num_cores, info.num_subcores
  NW = NC * NS
  b_per_w = B // NW
  mesh = plsc.VectorSubcoreMesh(core_axis_name="c", subcore_axis_name="s")

  @functools.partial(
      pl.kernel, mesh=mesh,
      out_type=jax.ShapeDtypeStruct((B, Dd), jnp.float32),
      scratch_types=[
          pltpu.VMEM((b_per_w,), jnp.int32),
          pltpu.VMEM((b_per_w, Dd), jnp.float32),
          pltpu.SemaphoreType.DMA,
      ],
  )
  def k(table_hbm, idx_hbm, out_hbm, idx_v, rows_v, sem):
    wid = lax.axis_index("s") * NC + lax.axis_index("c")
    base = wid * b_per_w
    pltpu.sync_copy(idx_hbm.at[pl.ds(base, b_per_w)], idx_v)
    pltpu.async_copy(table_hbm.at[idx_v], rows_v, sem).wait()
    pltpu.sync_copy(rows_v, out_hbm.at[pl.ds(base, b_per_w)])

  return k(table, idx)


def _ssm_body(x_ref, dw_ref, db_ref, bw_ref, bb_ref, cw_ref, cb_ref,
              alt_ref, dp_ref, rw_ref, rb_ref,
              out_ref, tw_ref, ti_ref,
              delta_s, bm_s, cm_s, ba_s, bb_s, hs_s):
  x = x_ref[...]
  delta_s[...] = jax.nn.softplus(
      jnp.dot(x, dw_ref[...], preferred_element_type=jnp.float32) + db_ref[...])
  bm_s[...] = jnp.dot(x, bw_ref[...], preferred_element_type=jnp.float32) + bb_ref[...]
  cm_s[...] = jnp.dot(x, cw_ref[...], preferred_element_type=jnp.float32) + cb_ref[...]
  a_t = -jnp.exp(alt_ref[...])

  def chunk_body(c, h):
    t0 = c * CHUNK
    dch = delta_s[pl.ds(t0, CHUNK), :]
    bch = bm_s[pl.ds(t0, CHUNK), :]
    xch = x_ref[pl.ds(t0, CHUNK), :]
    ba_s[...] = jnp.exp(jnp.minimum(dch[:, None, :] * a_t[None, :, :], 2.0))
    bb_s[...] = jnp.clip(dch[:, None, :] * bch[:, :, None], -2.0, 2.0) * xch[:, None, :]

    def step(t, hc):
      hc = ba_s[t] * hc + bb_s[t]
      hc = jnp.clip(hc, -100.0, 100.0)
      hs_s[t] = hc
      return hc

    h = lax.fori_loop(0, CHUNK, step, h)
    cch = cm_s[pl.ds(t0, CHUNK), :]
    y = jnp.sum(hs_s[...] * cch[:, :, None], axis=1)
    out_ref[pl.ds(t0, CHUNK), :] = y + xch * dp_ref[...]
    return h

  h0 = jnp.zeros((SS, D), jnp.float32)
  lax.fori_loop(0, L // CHUNK, chunk_body, h0)

  so = out_ref[...]
  logits = jnp.dot(so, rw_ref[...], preferred_element_type=jnp.float32) + rb_ref[...]
  m = jnp.max(logits, axis=1, keepdims=True)
  s = jnp.sum(jnp.exp(logits - m), axis=1, keepdims=True)
  tw_ref[...] = 1.0 / s
  lane = lax.broadcasted_iota(jnp.int32, logits.shape, 1)
  ti_ref[...] = jnp.min(jnp.where(logits == m, lane, E), axis=1, keepdims=True)


def _ssm_call(x, p):
  out_shapes = (
      jax.ShapeDtypeStruct((L, D), jnp.float32),
      jax.ShapeDtypeStruct((L, 1), jnp.float32),
      jax.ShapeDtypeStruct((L, 1), jnp.int32),
  )
  scratch = [
      pltpu.VMEM((L, D), jnp.float32),
      pltpu.VMEM((L, SS), jnp.float32),
      pltpu.VMEM((L, SS), jnp.float32),
      pltpu.VMEM((CHUNK, SS, D), jnp.float32),
      pltpu.VMEM((CHUNK, SS, D), jnp.float32),
      pltpu.VMEM((CHUNK, SS, D), jnp.float32),
  ]
  return pl.pallas_call(
      _ssm_body,
      out_shape=out_shapes,
      scratch_shapes=scratch,
  )(x, p['delta_w'], p['delta_b'].reshape(1, D),
    p['B_w'], p['B_b'].reshape(1, SS),
    p['C_w'], p['C_b'].reshape(1, SS),
    p['A_log'].T, p['Dp'].reshape(1, D),
    p['router_w'], p['router_b'].reshape(1, E))


def _split3(x):
  h1 = x.astype(jnp.bfloat16)
  r1 = x - h1.astype(jnp.float32)
  h2 = r1.astype(jnp.bfloat16)
  h3 = (r1 - h2.astype(jnp.float32)).astype(jnp.bfloat16)
  return h1, h2, h3


def _onehot_gather_exact(p_bf, x):
  h1, h2, h3 = _split3(x)
  d1 = jnp.dot(p_bf, h1, preferred_element_type=jnp.float32)
  d2 = jnp.dot(p_bf, h2, preferred_element_type=jnp.float32)
  d3 = jnp.dot(p_bf, h3, preferred_element_type=jnp.float32)
  return (d1 + d2) + d3


def _onehot_scatter_exact(p_bf, x):
  h1, h2, h3 = _split3(x)
  dn = (((0,), (0,)), ((), ()))
  d1 = lax.dot_general(p_bf, h1, dn, preferred_element_type=jnp.float32)
  d2 = lax.dot_general(p_bf, h2, dn, preferred_element_type=jnp.float32)
  d3 = lax.dot_general(p_bf, h3, dn, preferred_element_type=jnp.float32)
  return (d1 + d2) + d3


def _build_onehot(perm_ref, base, base_c, end):
  perm_sl = perm_ref[pl.ds(base_c, TT), :]
  sub = lax.broadcasted_iota(jnp.int32, (TT, 1), 0)
  pos = base_c + sub
  valid = (pos >= base) & (pos < end)
  lane = lax.broadcasted_iota(jnp.int32, (TT, L), 1)
  return jnp.where(valid & (perm_sl == lane), 1.0, 0.0).astype(jnp.bfloat16)


def _moe_body(offs_ref, flat_ref, perm_ref, tw_ref,
              uw_ref, ub_ref, dw_ref, db_ref, g_ref, b_ref,
              out_ref, xg_s, wg_s, yacc_s):
  e = pl.program_id(0)
  f = pl.program_id(1)
  j = pl.program_id(2)
  base = offs_ref[e] + j * TT
  end = offs_ref[e + 1]
  active = base < end
  base_c = jnp.minimum(base, L - TT)

  @pl.when((e == 0) & (f == 0) & (j == 0))
  def _():
    out_ref[...] = jnp.zeros_like(out_ref)

  @pl.when(active & (f == 0))
  def _():
    p_bf = _build_onehot(perm_ref, base, base_c, end)
    xg_s[j] = _onehot_gather_exact(p_bf, flat_ref[...])
    wg_s[j] = _onehot_gather_exact(p_bf, tw_ref[...])

  @pl.when(active)
  def _():
    hid = jnp.dot(xg_s[j], uw_ref[0], preferred_element_type=jnp.float32) + ub_ref[0]
    hid = hid * jax.nn.sigmoid(hid)
    part = jnp.dot(hid, dw_ref[0], preferred_element_type=jnp.float32)

    @pl.when(f == 0)
    def _():
      yacc_s[j] = part

    @pl.when(f != 0)
    def _():
      yacc_s[j] = yacc_s[j] + part

  @pl.when(active & (f == NF - 1))
  def _():
    ysel = (yacc_s[j] + db_ref[0]).astype(jnp.bfloat16).astype(jnp.float32)
    rows = xg_s[j] + ysel * wg_s[j]
    mu = jnp.mean(rows, axis=1, keepdims=True)
    dev = rows - mu
    var = jnp.mean(dev * dev, axis=1, keepdims=True)
    ln = dev / jnp.sqrt(var + 1e-5) * g_ref[...] + b_ref[...]
    p_bf = _build_onehot(perm_ref, base, base_c, end)
    out_ref[...] = out_ref[...] + _onehot_scatter_exact(p_bf, ln)


def _moe_call(flat, perm, offs, tw, p):
  grid = (E, NF, NT)
  return pl.pallas_call(
      _moe_body,
      grid=grid,
      in_specs=[
          pl.BlockSpec(memory_space=pltpu.SMEM),
          pl.BlockSpec((L, D), lambda e, f, j: (0, 0)),
          pl.BlockSpec((L, 1), lambda e, f, j: (0, 0)),
          pl.BlockSpec((L, 1), lambda e, f, j: (0, 0)),
          pl.BlockSpec((1, D, FF), lambda e, f, j: (e, 0, f)),
          pl.BlockSpec((1, 1, FF), lambda e, f, j: (e, 0, f)),
          pl.BlockSpec((1, FF, D), lambda e, f, j: (e, f, 0)),
          pl.BlockSpec((1, 1, D), lambda e, f, j: (e, 0, 0)),
          pl.BlockSpec((1, D), lambda e, f, j: (0, 0)),
          pl.BlockSpec((1, D), lambda e, f, j: (0, 0)),
      ],
      out_specs=pl.BlockSpec((L, D), lambda e, f, j: (0, 0)),
      out_shape=jax.ShapeDtypeStruct((L, D), jnp.float32),
      scratch_shapes=[
          pltpu.VMEM((NT, TT, D), jnp.float32),
          pltpu.VMEM((NT, TT, 1), jnp.float32),
          pltpu.VMEM((NT, TT, D), jnp.float32),
      ],
  )(offs, flat, perm, tw,
    p['up_w'], p['up_b'].reshape(E, 1, DFF), p['down_w'],
    p['down_b'].reshape(E, 1, D),
    p['ln_g'].reshape(1, D), p['ln_b'].reshape(1, D))


def _unembed_body(h_ref, emb_ref, out_ref):
  out_ref[...] = lax.dot_general(
      h_ref[...], emb_ref[...], (((1,), (1,)), ((), ())),
      preferred_element_type=jnp.float32)


def _unembed_call(h, embed):
  V = embed.shape[0]
  return pl.pallas_call(
      _unembed_body,
      grid=(V // VT,),
      in_specs=[
          pl.BlockSpec((L, D), lambda v: (0, 0)),
          pl.BlockSpec((VT, D), lambda v: (v, 0)),
      ],
      out_specs=pl.BlockSpec((L, VT), lambda v: (0, v)),
      out_shape=jax.ShapeDtypeStruct((L, V), jnp.float32),
  )(h, embed)


def _layer(lp, h):
  ssm_out, tw, ti = _ssm_call(h, lp)
  ti_flat = ti.reshape(L)
  perm = jnp.argsort(ti_flat).astype(jnp.int32)
  counts = jnp.bincount(ti_flat, length=E)
  offs = jnp.concatenate([jnp.zeros((1,), jnp.int32),
                          jnp.cumsum(counts).astype(jnp.int32)])
  return _moe_call(ssm_out, perm.reshape(L, 1), offs, tw, lp)


def kernel(x, params):
  idx = x.reshape(L).astype(jnp.int32)
  h = _emb_gather_sc(params['embed'], idx)
  for lp in params['layers']:
    h = _layer(lp, h)
  logits = _unembed_call(h, params['embed'])
  return logits.reshape(1, L, -1)
